# baseline (device time: 48552 ns/iter reference)
import jax
import jax.numpy as jnp
from jax import lax
from jax.experimental import pallas as pl
from jax.experimental.pallas import tpu as pltpu

N_DEV = 8
SQ = 1024
D = 1024
HQ = 8
DH = 128
BLK = 64
SCALE = 0.08838834764831843
CBIG = 256
N_CBIG = SQ // CBIG

COMM = [(0, 128), (128, 128), (256, 128), (384, 128), (512, 128),
        (640, 128), (768, 64), (832, 64), (896, 64), (960, 64)]
CBIG_COMM = [(0, 1), (2, 3), (4, 5), (6, 7, 8, 9)]
N_COMM = len(COMM)

_MESH = pl.DeviceIdType.MESH


def kernel(x, Wq, K_ext, V_ext, Wo):
    bf16 = jnp.bfloat16
    x2 = x.reshape(SQ, D)
    k16 = K_ext.reshape(SQ, HQ * DH).astype(bf16)
    v16 = V_ext.reshape(SQ, HQ * DH).astype(bf16)

    def body(x_ref, wq_ref, k_ref, v_ref, wo_ref, out_ref,
             ctx_ref, wq16_ref, wo16_ref, q16_ref,
             ready_a, ready_b, ready_c, send_sems, far_b_sems, far_c_sems,
             recv_sems):
        my = lax.axis_index("i")
        is_src = my == 0
        prev = jnp.where(my == 2, 1,
               jnp.where(my == 5, 4,
               jnp.where(my == 6, 5,
               jnp.where(my == 7, 3, 0))))
        nxt = jnp.where(my == 1, 2,
              jnp.where(my == 3, 7,
              jnp.where(my == 4, 5,
              jnp.where(my == 5, 6, 0))))
        is_fwd = jnp.logical_or(
            jnp.logical_or(my == 1, my == 3),
            jnp.logical_or(my == 4, my == 5))
        has_recv = my != 0

        @pl.when(my == 3)
        def _():
            pl.semaphore_signal(ready_b, inc=1, device_id=(0,),
                                device_id_type=_MESH)

        @pl.when(my == 4)
        def _():
            pl.semaphore_signal(ready_c, inc=1, device_id=(0,),
                                device_id_type=_MESH)

        @pl.when(jnp.logical_and(has_recv,
                                 jnp.logical_and(my != 3, my != 4)))
        def _():
            pl.semaphore_signal(ready_a, inc=1, device_id=(prev,),
                                device_id_type=_MESH)

        @pl.when(is_src)
        def _():
            pl.semaphore_wait(ready_a, 1)
            pl.semaphore_wait(ready_b, 1)
            pl.semaphore_wait(ready_c, 1)

        @pl.when(is_fwd)
        def _():
            pl.semaphore_wait(ready_a, 1)

        wo16_ref[...] = wo_ref[...].astype(bf16)

        @pl.when(is_src)
        def _():
            wq16_ref[...] = wq_ref[...].astype(bf16)
            q = jnp.dot(x_ref[...].astype(bf16), wq16_ref[...],
                        preferred_element_type=jnp.float32)
            q16_ref[...] = (q * SCALE).astype(bf16)

        def compute_ctx_chunk(c2):
            L = (c2 + 1) * CBIG
            sl = pl.ds(c2 * CBIG, CBIG)
            q16 = q16_ref[sl, :]
            ri = lax.broadcasted_iota(jnp.int32, (CBIG, L), 0)
            ci = lax.broadcasted_iota(jnp.int32, (CBIG, L), 1)
            mask = (ci // BLK) <= (ri // BLK + 4 * c2)
            for h in range(HQ):
                hs = slice(h * DH, (h + 1) * DH)
                qh = q16[:, hs]
                kh = k_ref[0:L, hs]
                vh = v_ref[0:L, hs]
                s = lax.dot_general(qh, kh, (((1,), (1,)), ((), ())),
                                    preferred_element_type=jnp.float32)
                w = jnp.where(mask, jnp.exp(s), 0.0)
                wsum = jnp.sum(w, axis=1, keepdims=True)
                ctx = jnp.dot(w.astype(bf16), vh,
                              preferred_element_type=jnp.float32)
                ctx_ref[sl, hs] = (ctx / wsum).astype(bf16)

        def chunk_desc(c, sems, target):
            lo, rows = COMM[c]
            sl = pl.ds(lo, rows)
            return pltpu.make_async_remote_copy(
                src_ref=ctx_ref.at[sl, :],
                dst_ref=ctx_ref.at[sl, :],
                send_sem=sems.at[c],
                recv_sem=recv_sems.at[c],
                device_id=(target,),
                device_id_type=_MESH,
            )

        def wo_rows(lo, rows):
            sl = pl.ds(lo, rows)
            out_ref[0, sl, :] = jnp.dot(ctx_ref[sl, :], wo16_ref[...],
                                        preferred_element_type=jnp.float32)

        for c2 in range(N_CBIG):
            @pl.when(is_src)
            def _(c2=c2):
                compute_ctx_chunk(c2)
                for c in CBIG_COMM[c2]:
                    chunk_desc(c, send_sems, 1).start()
                    chunk_desc(c, far_b_sems, 3).start()
                    chunk_desc(c, far_c_sems, 4).start()

            for c in CBIG_COMM[c2]:
                @pl.when(has_recv)
                def _(c=c):
                    chunk_desc(c, send_sems, nxt).wait_recv()

                @pl.when(is_fwd)
                def _(c=c):
                    chunk_desc(c, send_sems, nxt).start()

            @pl.when(has_recv)
            def _(c2=c2):
                wo_rows(c2 * CBIG, CBIG)

        @pl.when(is_src)
        def _():
            wo_rows(0, SQ)

        for c in range(N_COMM):
            @pl.when(is_src)
            def _(c=c):
                chunk_desc(c, send_sems, 1).wait_send()
                chunk_desc(c, far_b_sems, 3).wait_send()
                chunk_desc(c, far_c_sems, 4).wait_send()

            @pl.when(is_fwd)
            def _(c=c):
                chunk_desc(c, send_sems, nxt).wait_send()

    out = pl.pallas_call(
        body,
        out_shape=jax.ShapeDtypeStruct((1, SQ, D), jnp.float32),
        in_specs=[pl.BlockSpec(memory_space=pltpu.VMEM)] * 5,
        out_specs=pl.BlockSpec(memory_space=pltpu.VMEM),
        scratch_shapes=[
            pltpu.VMEM((SQ, D), jnp.bfloat16),
            pltpu.VMEM((D, D), jnp.bfloat16),
            pltpu.VMEM((D, D), jnp.bfloat16),
            pltpu.VMEM((SQ, D), jnp.bfloat16),
            pltpu.SemaphoreType.REGULAR,
            pltpu.SemaphoreType.REGULAR,
            pltpu.SemaphoreType.REGULAR,
            pltpu.SemaphoreType.DMA((N_COMM,)),
            pltpu.SemaphoreType.DMA((N_COMM,)),
            pltpu.SemaphoreType.DMA((N_COMM,)),
            pltpu.SemaphoreType.DMA((N_COMM,)),
        ],
    )(x2, Wq, k16, v16, Wo)
    return out


# device time: 46766 ns/iter; 1.0382x vs baseline; 1.0382x over previous
import jax
import jax.numpy as jnp
from jax import lax
from jax.experimental import pallas as pl
from jax.experimental.pallas import tpu as pltpu

N_DEV = 8
SQ = 1024
D = 1024
HQ = 8
DH = 128
BLK = 64
SCALE = 0.08838834764831843
CBIG = 256
N_CBIG = SQ // CBIG

COMM = [(0, 128), (128, 128), (256, 128), (384, 128), (512, 128),
        (640, 128), (768, 64), (832, 64), (896, 64), (960, 64)]
CBIG_COMM = [(0, 1), (2, 3), (4, 5), (6, 7, 8, 9)]
N_COMM = len(COMM)

_MESH = pl.DeviceIdType.MESH


def kernel(x, Wq, K_ext, V_ext, Wo):
    bf16 = jnp.bfloat16
    x2 = x.reshape(SQ, D)
    wo16 = Wo.astype(bf16)
    k16 = K_ext.reshape(SQ, HQ * DH).astype(bf16)
    v16 = V_ext.reshape(SQ, HQ * DH).astype(bf16)

    def body(x_ref, wq_ref, k_ref, v_ref, wo_ref, out_ref,
             ctx_ref, wq16_ref,
             ready_a, ready_b, ready_c, send_sems, far_b_sems, far_c_sems,
             recv_sems):
        my = lax.axis_index("i")
        is_src = my == 0
        prev = jnp.where(my == 2, 1,
               jnp.where(my == 5, 4,
               jnp.where(my == 6, 5,
               jnp.where(my == 7, 3, 0))))
        nxt = jnp.where(my == 1, 2,
              jnp.where(my == 3, 7,
              jnp.where(my == 4, 5,
              jnp.where(my == 5, 6, 0))))
        is_fwd = jnp.logical_or(
            jnp.logical_or(my == 1, my == 3),
            jnp.logical_or(my == 4, my == 5))
        has_recv = my != 0

        @pl.when(my == 3)
        def _():
            pl.semaphore_signal(ready_b, inc=1, device_id=(0,),
                                device_id_type=_MESH)

        @pl.when(my == 4)
        def _():
            pl.semaphore_signal(ready_c, inc=1, device_id=(0,),
                                device_id_type=_MESH)

        @pl.when(jnp.logical_and(has_recv,
                                 jnp.logical_and(my != 3, my != 4)))
        def _():
            pl.semaphore_signal(ready_a, inc=1, device_id=(prev,),
                                device_id_type=_MESH)

        @pl.when(is_src)
        def _():
            pl.semaphore_wait(ready_a, 1)
            pl.semaphore_wait(ready_b, 1)
            pl.semaphore_wait(ready_c, 1)

        @pl.when(is_fwd)
        def _():
            pl.semaphore_wait(ready_a, 1)

        @pl.when(is_src)
        def _():
            wq16_ref[...] = wq_ref[...].astype(bf16)

        def compute_ctx_chunk(c2):
            L = (c2 + 1) * CBIG
            sl = pl.ds(c2 * CBIG, CBIG)
            q = jnp.dot(x_ref[sl, :].astype(bf16), wq16_ref[...],
                        preferred_element_type=jnp.float32)
            q16 = (q * SCALE).astype(bf16)
            ri = lax.broadcasted_iota(jnp.int32, (CBIG, L), 0)
            ci = lax.broadcasted_iota(jnp.int32, (CBIG, L), 1)
            mask = (ci // BLK) <= (ri // BLK + 4 * c2)
            for h in range(HQ):
                hs = slice(h * DH, (h + 1) * DH)
                qh = q16[:, hs]
                kh = k_ref[0:L, hs]
                vh = v_ref[0:L, hs]
                s = lax.dot_general(qh, kh, (((1,), (1,)), ((), ())),
                                    preferred_element_type=jnp.float32)
                w = jnp.where(mask, jnp.exp(s), 0.0)
                wsum = jnp.sum(w, axis=1, keepdims=True)
                ctx = jnp.dot(w.astype(bf16), vh,
                              preferred_element_type=jnp.float32)
                ctx_ref[sl, hs] = (ctx / wsum).astype(bf16)

        def chunk_desc(c, sems, target):
            lo, rows = COMM[c]
            sl = pl.ds(lo, rows)
            return pltpu.make_async_remote_copy(
                src_ref=ctx_ref.at[sl, :],
                dst_ref=ctx_ref.at[sl, :],
                send_sem=sems.at[c],
                recv_sem=recv_sems.at[c],
                device_id=(target,),
                device_id_type=_MESH,
            )

        def wo_rows(lo, rows):
            sl = pl.ds(lo, rows)
            out_ref[0, sl, :] = jnp.dot(ctx_ref[sl, :], wo_ref[...],
                                        preferred_element_type=jnp.float32)

        for c2 in range(N_CBIG):
            @pl.when(is_src)
            def _(c2=c2):
                compute_ctx_chunk(c2)
                for c in CBIG_COMM[c2]:
                    chunk_desc(c, send_sems, 1).start()
                    chunk_desc(c, far_b_sems, 3).start()
                    chunk_desc(c, far_c_sems, 4).start()

            for c in CBIG_COMM[c2]:
                @pl.when(has_recv)
                def _(c=c):
                    chunk_desc(c, send_sems, nxt).wait_recv()

                @pl.when(is_fwd)
                def _(c=c):
                    chunk_desc(c, send_sems, nxt).start()

            @pl.when(has_recv)
            def _(c2=c2):
                wo_rows(c2 * CBIG, CBIG)

        @pl.when(is_src)
        def _():
            wo_rows(0, SQ)

        for c in range(N_COMM):
            @pl.when(is_src)
            def _(c=c):
                chunk_desc(c, send_sems, 1).wait_send()
                chunk_desc(c, far_b_sems, 3).wait_send()
                chunk_desc(c, far_c_sems, 4).wait_send()

            @pl.when(is_fwd)
            def _(c=c):
                chunk_desc(c, send_sems, nxt).wait_send()

    out = pl.pallas_call(
        body,
        out_shape=jax.ShapeDtypeStruct((1, SQ, D), jnp.float32),
        in_specs=[pl.BlockSpec(memory_space=pltpu.VMEM)] * 5,
        out_specs=pl.BlockSpec(memory_space=pltpu.VMEM),
        scratch_shapes=[
            pltpu.VMEM((SQ, D), jnp.bfloat16),
            pltpu.VMEM((D, D), jnp.bfloat16),
            pltpu.SemaphoreType.REGULAR,
            pltpu.SemaphoreType.REGULAR,
            pltpu.SemaphoreType.REGULAR,
            pltpu.SemaphoreType.DMA((N_COMM,)),
            pltpu.SemaphoreType.DMA((N_COMM,)),
            pltpu.SemaphoreType.DMA((N_COMM,)),
            pltpu.SemaphoreType.DMA((N_COMM,)),
        ],
    )(x2, Wq, k16, v16, wo16)
    return out
